# baseline (speedup 1.0000x reference)
import functools

import numpy as np

import jax
import jax.numpy as jnp
from jax import lax
from jax.experimental import pallas as pl
from jax.experimental.pallas import tpu as pltpu
from jax.experimental.pallas import tpu_sc as plsc

N_NODES = 10000
N_EDGES = 160000
D_IN = 256
D_OUT = 256
DW = D_OUT // 2
NC, NS, L = 2, 16, 16
N_PAD = 10240
NBLK = 4
BLK = N_PAD // NBLK
ROWS_CP = BLK // NS
RB_CP = 40
E_PAD = 163840
E_PER_TILE = E_PAD // NS
SCN = 1024
NSCN = E_PER_TILE // SCN
CAP = 3072
K = 64


def _mm_body(x_ref, wt_ref, b_ref, o_ref):
    o_ref[...] = (jnp.dot(x_ref[...], wt_ref[...],
                          preferred_element_type=jnp.float32)
                  + b_ref[...]).astype(jnp.bfloat16)


def _linear_bf16(x, Wt, b2):
    RB = 1000
    return pl.pallas_call(
        _mm_body,
        grid=(N_NODES // RB,),
        in_specs=[
            pl.BlockSpec((RB, D_IN), lambda i: (i, 0)),
            pl.BlockSpec((D_IN, D_OUT), lambda i: (0, 0)),
            pl.BlockSpec((1, D_OUT), lambda i: (0, 0)),
        ],
        out_specs=pl.BlockSpec((RB, D_OUT), lambda i: (i, 0)),
        out_shape=jax.ShapeDtypeStruct((N_NODES, D_OUT), jnp.bfloat16),
    )(x, Wt, b2)


def _sc_aggregate(h32, e3):
    mesh = plsc.VectorSubcoreMesh(core_axis_name="c", subcore_axis_name="s")

    @functools.partial(
        pl.kernel,
        out_type=jax.ShapeDtypeStruct((N_PAD, D_OUT), jnp.float32),
        mesh=mesh,
        scratch_types=[
            pltpu.VMEM((3, SCN), jnp.int32),
            pltpu.VMEM((CAP,), jnp.int32),
            pltpu.VMEM((CAP,), jnp.int32),
            pltpu.VMEM((CAP,), jnp.int32),
            pltpu.VMEM((CAP,), jnp.int32),
            pltpu.VMEM((CAP,), jnp.float32),
            pltpu.VMEM((CAP,), jnp.float32),
            pltpu.VMEM((K, DW), jnp.int32),
            pltpu.VMEM((K, DW), jnp.int32),
            pltpu.VMEM((K, D_OUT), jnp.float32),
            pltpu.VMEM((K,), jnp.int32),
            pltpu.VMEM((RB_CP, D_OUT), jnp.float32),
            pltpu.VMEM_SHARED((BLK, D_OUT), jnp.float32),
            pltpu.SemaphoreType.DMA,
            pltpu.SemaphoreType.DMA,
        ],
        compiler_params=pltpu.CompilerParams(use_tc_tiling_on_sc=False,
                                            needs_layout_passes=False),
    )
    def k(h_hbm, e3_hbm, o_hbm,
          scan_v, srcc0, srcc1, dstc0, dstc1, wc0, wc1,
          gbuf0, gbuf1, sbuf, dref,
          buf_v, acc_sh, gsem0, gsem1):
        cblk = ((srcc0, dstc0, wc0), (srcc1, dstc1, wc1))
        c = lax.axis_index("c")
        s = lax.axis_index("s")

        def zrow(i, carry):
            for j in range(D_OUT // L):
                buf_v[i, pl.ds(j * L, L)] = jnp.zeros((L,), jnp.float32)
            return carry
        lax.fori_loop(0, RB_CP, zrow, 0)

        zi = jnp.zeros((L,), jnp.int32)
        zf = jnp.zeros((L,), jnp.float32)

        def zcap(i, carry):
            for b in range(2):
                cblk[b][0][pl.ds(i * L, L)] = zi
                cblk[b][1][pl.ds(i * L, L)] = zi
                cblk[b][2][pl.ds(i * L, L)] = zf
            return carry
        lax.fori_loop(0, CAP // L, zcap, 0)

        lo0 = (2 * c) * BLK
        lo1 = (2 * c + 1) * BLK

        def scan_chunk(ch, offs):
            pltpu.sync_copy(e3_hbm.at[s, ch], scan_v)

            def group(g, offs):
                off0, off1 = offs
                sl = pl.ds(g * L, L)
                src16 = scan_v[0, sl]
                dst16 = scan_v[1, sl]
                w16 = plsc.bitcast(scan_v[2, sl], jnp.float32)
                for b, lo in ((0, lo0), (1, lo1)):
                    off = off0 if b == 0 else off1
                    m = (dst16 >= lo) & (dst16 < lo + BLK)
                    cs = plsc.cumsum(m.astype(jnp.int32))
                    pos = jnp.minimum(cs + (off - 1), CAP - 1)
                    plsc.store_scatter(cblk[b][0], [pos], src16, mask=m)
                    plsc.store_scatter(cblk[b][1], [pos], dst16 - lo, mask=m)
                    plsc.store_scatter(cblk[b][2], [pos], w16, mask=m)
                    off = jnp.minimum(off + cs[L - 1], CAP)
                    if b == 0:
                        off0 = off
                    else:
                        off1 = off
                return (off0, off1)
            return lax.fori_loop(0, SCN // L, group, offs)

        cnt0, cnt1 = lax.fori_loop(0, NSCN, scan_chunk, (jnp.int32(0),
                                                         jnp.int32(0)))

        def process(bI, t, gbuf, sbuf, dref):
            def mul(eg, cc):
                wv16 = cblk[bI][2][pl.ds(t * K + eg * L, L)]
                for e in range(L):
                    wb = wv16[e]
                    i = eg * L + e
                    for j in range(DW // L):
                        u = gbuf[i, pl.ds(j * L, L)]
                        ev = plsc.bitcast(u << 16, jnp.float32)
                        od = plsc.bitcast(u & jnp.int32(-65536), jnp.float32)
                        sbuf[i, pl.ds(2 * j * L, L)] = ev * wb
                        sbuf[i, pl.ds((2 * j + 1) * L, L)] = od * wb
                return cc
            lax.fori_loop(0, K // L, mul, 0)
            for r in range(K // L):
                sl = pl.ds(r * L, L)
                dref[sl] = cblk[bI][1][pl.ds(t * K + r * L, L)]

        def pipeline(bI, cnt):
            blk = 2 * c + bI
            nch2 = (cnt + 2 * K - 1) // (2 * K)
            nch = 2 * nch2
            r0 = s * ROWS_CP

            def zhop(i, carry):
                pltpu.sync_copy(buf_v, acc_sh.at[pl.ds(r0 + i * RB_CP, RB_CP)])
                return carry
            lax.fori_loop(0, ROWS_CP // RB_CP, zhop, 0)
            plsc.subcore_barrier()

            def gslice(t):
                return cblk[bI][0].at[pl.ds(t * K, K)]

            pltpu.async_copy(h_hbm.at[gslice(0)], gbuf0, gsem0)
            pltpu.async_copy(h_hbm.at[gslice(1)], gbuf1, gsem1)

            def stage(t, gbuf, gsem):
                pltpu.make_async_copy(h_hbm.at[gslice(t)], gbuf, gsem).wait()
                process(bI, t, gbuf, sbuf, dref)

                @pl.when(t + 2 < nch)
                def _():
                    pltpu.async_copy(h_hbm.at[gslice(t + 2)], gbuf, gsem)
                pltpu.sync_copy(sbuf, acc_sh.at[dref], add=True)

            def pair(u, carry):
                t0 = 2 * u
                stage(t0, gbuf0, gsem0)
                stage(t0 + 1, gbuf1, gsem1)
                return carry
            lax.fori_loop(0, nch2, pair, 0)
            plsc.subcore_barrier()

            def cphop(i, carry):
                pltpu.sync_copy(acc_sh.at[pl.ds(r0 + i * RB_CP, RB_CP)], buf_v)
                pltpu.sync_copy(
                    buf_v, o_hbm.at[pl.ds(blk * BLK + r0 + i * RB_CP, RB_CP)])
                return carry
            lax.fori_loop(0, ROWS_CP // RB_CP, cphop, 0)
            lax.fori_loop(0, RB_CP, zrow, 0)

        pipeline(0, cnt0)
        pipeline(1, cnt1)

    return k(h32, e3)


def kernel(x, edge_index, edge_weight, W, b):
    src = edge_index[0].astype(jnp.int32)
    dst = edge_index[1].astype(jnp.int32)
    pad = E_PAD - N_EDGES
    src_p = jnp.concatenate([src, jnp.zeros((pad,), jnp.int32)])
    dst_p = jnp.concatenate(
        [dst, (jnp.arange(pad, dtype=jnp.int32) % NBLK) * BLK])
    w_p = jnp.concatenate([edge_weight, jnp.zeros((pad,), jnp.float32)])
    e3 = jnp.stack(
        [src_p.reshape(NS, NSCN, SCN),
         dst_p.reshape(NS, NSCN, SCN),
         lax.bitcast_convert_type(w_p, jnp.int32).reshape(NS, NSCN, SCN)],
        axis=2)

    h = _linear_bf16(x, W.T, b.reshape(1, D_OUT))
    h32 = lax.bitcast_convert_type(
        h.reshape(N_NODES, DW, 2), jnp.int32)

    o = _sc_aggregate(h32, e3)
    n = np.arange(D_OUT)
    col = 32 * (n // 32) + (n % 32) // 2 + 16 * (n % 2)
    return o[:N_NODES, jnp.asarray(col)]


# speedup vs baseline: 4.3932x; 4.3932x over previous
import functools

import numpy as np

import jax
import jax.numpy as jnp
from jax import lax
from jax.experimental import pallas as pl
from jax.experimental.pallas import tpu as pltpu
from jax.experimental.pallas import tpu_sc as plsc

N_NODES = 10000
N_EDGES = 160000
D_IN = 256
D_OUT = 256
DQ = 64
NC, NS, L = 2, 16, 16
K = 128
E_PAD = 163840
E_PER_TILE = E_PAD // NS
NCHUNK = E_PER_TILE // K
NPAIR = NCHUNK // 2
N_PAD = 10240
ROWS_PER_TILE = N_PAD // NS
RB_CP = 128
N_CP = ROWS_PER_TILE // RB_CP


def _mm_body(x_ref, wt_ref, b_ref, o_ref):
    o_ref[0] = (jnp.dot(x_ref[...], wt_ref[0],
                        preferred_element_type=jnp.float32)
                + b_ref[0]).astype(jnp.bfloat16)


def _linear_quarters(x, Wt4, b4):
    RB = 1000
    return pl.pallas_call(
        _mm_body,
        grid=(N_NODES // RB, 4),
        in_specs=[
            pl.BlockSpec((RB, D_IN), lambda i, j: (i, 0)),
            pl.BlockSpec((1, D_IN, DQ), lambda i, j: (j, 0, 0)),
            pl.BlockSpec((1, 1, DQ), lambda i, j: (j, 0, 0)),
        ],
        out_specs=pl.BlockSpec((1, RB, DQ), lambda i, j: (j, i, 0)),
        out_shape=jax.ShapeDtypeStruct((4, N_NODES, DQ), jnp.bfloat16),
    )(x, Wt4, b4)


def _sc_aggregate(h4w, src3, dst3, w2):
    mesh = plsc.VectorSubcoreMesh(core_axis_name="c", subcore_axis_name="s")

    @functools.partial(
        pl.kernel,
        out_type=[jax.ShapeDtypeStruct((N_PAD, DQ), jnp.float32)] * 4,
        mesh=mesh,
        scratch_types=[
            pltpu.VMEM((NCHUNK, K), jnp.int32),
            pltpu.VMEM((NCHUNK, K), jnp.int32),
            pltpu.VMEM((E_PER_TILE,), jnp.float32),
            pltpu.VMEM((K, DQ // 2), jnp.int32),
            pltpu.VMEM((K, DQ // 2), jnp.int32),
            pltpu.VMEM((K, DQ), jnp.float32),
            pltpu.VMEM((K, DQ), jnp.float32),
            pltpu.VMEM((RB_CP, DQ), jnp.float32),
            pltpu.VMEM_SHARED((N_PAD, DQ), jnp.float32),
            pltpu.SemaphoreType.DMA,
            pltpu.SemaphoreType.DMA,
            pltpu.SemaphoreType.DMA,
            pltpu.SemaphoreType.DMA,
        ],
        compiler_params=pltpu.CompilerParams(
            use_tc_tiling_on_sc=False, needs_layout_passes=False),
    )
    def k(h0_hbm, h1_hbm, h2_hbm, h3_hbm, src_hbm, dst_hbm, w_hbm,
          o0_hbm, o1_hbm, o2_hbm, o3_hbm,
          src_v, dst_v, w_v, gbuf0, gbuf1, sbuf0, sbuf1, buf_v, acc_sh,
          gsem0, gsem1, ssem0, ssem1):
        c = lax.axis_index("c")
        s = lax.axis_index("s")
        r0 = s * ROWS_PER_TILE

        pltpu.sync_copy(src_hbm.at[s], src_v)
        pltpu.sync_copy(dst_hbm.at[s], dst_v)
        pltpu.sync_copy(w_hbm.at[s], w_v)

        def zrow(i, carry):
            for j in range(DQ // L):
                buf_v[i, pl.ds(j * L, L)] = jnp.zeros((L,), jnp.float32)
            return carry
        lax.fori_loop(0, RB_CP, zrow, 0)

        def process(t, gbuf, sbuf):
            def mul(g, cc):
                wv16 = w_v[pl.ds(t * K + g * L, L)]
                for e in range(L):
                    wb = wv16[e]
                    i = g * L + e
                    for j in range(DQ // (2 * L)):
                        u = gbuf[i, pl.ds(j * L, L)]
                        ev = plsc.bitcast(u << 16, jnp.float32)
                        od = plsc.bitcast(u & jnp.int32(-65536), jnp.float32)
                        sbuf[i, pl.ds(2 * j * L, L)] = ev * wb
                        sbuf[i, pl.ds((2 * j + 1) * L, L)] = od * wb
                return cc
            lax.fori_loop(0, K // L, mul, 0)

        def pipeline(h_q, o_q):
            def zhop(i, carry):
                pltpu.sync_copy(
                    buf_v, acc_sh.at[pl.ds(r0 + i * RB_CP, RB_CP)])
                return carry
            lax.fori_loop(0, N_CP, zhop, 0)
            plsc.subcore_barrier()

            pltpu.async_copy(h_q.at[src_v.at[0]], gbuf0, gsem0)
            pltpu.async_copy(h_q.at[src_v.at[1]], gbuf1, gsem1)

            def stage(t, gbuf, sbuf, gsem, ssem, u):
                pltpu.make_async_copy(
                    h_q.at[src_v.at[t]], gbuf, gsem).wait()

                @pl.when(u > 0)
                def _():
                    pltpu.make_async_copy(
                        sbuf, acc_sh.at[dst_v.at[t]], ssem).wait()
                process(t, gbuf, sbuf)

                @pl.when(t + 2 < NCHUNK)
                def _():
                    pltpu.async_copy(h_q.at[src_v.at[t + 2]], gbuf, gsem)
                pltpu.async_copy(sbuf, acc_sh.at[dst_v.at[t]], ssem, add=True)

            def pair(u, carry):
                t0 = 2 * u
                stage(t0, gbuf0, sbuf0, gsem0, ssem0, u)
                stage(t0 + 1, gbuf1, sbuf1, gsem1, ssem1, u)
                return carry
            lax.fori_loop(0, NPAIR, pair, 0)
            pltpu.make_async_copy(
                sbuf0, acc_sh.at[dst_v.at[NCHUNK - 2]], ssem0).wait()
            pltpu.make_async_copy(
                sbuf1, acc_sh.at[dst_v.at[NCHUNK - 1]], ssem1).wait()
            plsc.subcore_barrier()

            def cphop(i, carry):
                pltpu.sync_copy(
                    acc_sh.at[pl.ds(r0 + i * RB_CP, RB_CP)], buf_v)
                pltpu.sync_copy(
                    buf_v, o_q.at[pl.ds(r0 + i * RB_CP, RB_CP)])
                return carry
            lax.fori_loop(0, N_CP, cphop, 0)
            lax.fori_loop(0, RB_CP, zrow, 0)

        @pl.when(c == 0)
        def _():
            pipeline(h0_hbm, o0_hbm)
            pipeline(h1_hbm, o1_hbm)

        @pl.when(c == 1)
        def _():
            pipeline(h2_hbm, o2_hbm)
            pipeline(h3_hbm, o3_hbm)

    return k(h4w[0], h4w[1], h4w[2], h4w[3], src3, dst3, w2)


def kernel(x, edge_index, edge_weight, W, b):
    src = edge_index[0].astype(jnp.int32)
    dst = edge_index[1].astype(jnp.int32)
    pad = E_PAD - N_EDGES
    zi = jnp.zeros((pad,), jnp.int32)
    src3 = jnp.concatenate([src, zi]).reshape(NS, NCHUNK, K)
    dst3 = jnp.concatenate([dst, zi]).reshape(NS, NCHUNK, K)
    w2 = jnp.concatenate(
        [edge_weight, jnp.zeros((pad,), jnp.float32)]).reshape(NS, E_PER_TILE)
    wt4 = jnp.transpose(W.T.reshape(D_IN, 4, DQ), (1, 0, 2))
    h4 = _linear_quarters(x, wt4, b.reshape(4, 1, DQ))
    h4w = lax.bitcast_convert_type(
        h4.reshape(4, N_NODES, DQ // 2, 2), jnp.int32)
    outs = _sc_aggregate(h4w, src3, dst3, w2)
    o = jnp.concatenate([o[:N_NODES] for o in outs], axis=1)
    n = np.arange(D_OUT)
    col = 32 * (n // 32) + (n % 32) // 2 + 16 * (n % 2)
    return o[:, jnp.asarray(col)]


# speedup vs baseline: 4.9539x; 1.1276x over previous
import functools

import jax
import jax.numpy as jnp
from jax import lax
from jax.experimental import pallas as pl
from jax.experimental.pallas import tpu as pltpu
from jax.experimental.pallas import tpu_sc as plsc

N_NODES = 10000
N_EDGES = 160000
D_IN = 256
D_OUT = 256
DQ = 64
NC, NS, L = 2, 16, 16
K = 128
E_PAD = 163840
E_PER_TILE = E_PAD // NS
NCHUNK = E_PER_TILE // K
NPAIR = NCHUNK // 2
N_PAD = 10240
ROWS_PER_TILE = N_PAD // NS
RB_CP = 128
N_CP = ROWS_PER_TILE // RB_CP


def _mm_body(x_ref, wt_ref, b_ref, o_ref):
    o_ref[0] = jnp.dot(x_ref[...], wt_ref[0],
                       preferred_element_type=jnp.float32) + b_ref[0]


def _linear_quarters(x, Wt4, b4):
    RB = 1000
    return pl.pallas_call(
        _mm_body,
        grid=(N_NODES // RB, 4),
        in_specs=[
            pl.BlockSpec((RB, D_IN), lambda i, j: (i, 0)),
            pl.BlockSpec((1, D_IN, DQ), lambda i, j: (j, 0, 0)),
            pl.BlockSpec((1, 1, DQ), lambda i, j: (j, 0, 0)),
        ],
        out_specs=pl.BlockSpec((1, RB, DQ), lambda i, j: (j, i, 0)),
        out_shape=jax.ShapeDtypeStruct((4, N_NODES, DQ), jnp.float32),
    )(x, Wt4, b4)


def _sc_aggregate(h4, src3, dst3, w2):
    mesh = plsc.VectorSubcoreMesh(core_axis_name="c", subcore_axis_name="s")

    @functools.partial(
        pl.kernel,
        out_type=[jax.ShapeDtypeStruct((N_PAD, DQ), jnp.float32)] * 4,
        mesh=mesh,
        scratch_types=[
            pltpu.VMEM((NCHUNK, K), jnp.int32),
            pltpu.VMEM((NCHUNK, K), jnp.int32),
            pltpu.VMEM((E_PER_TILE,), jnp.float32),
            pltpu.VMEM((K, DQ), jnp.float32),
            pltpu.VMEM((K, DQ), jnp.float32),
            pltpu.VMEM((K, DQ), jnp.float32),
            pltpu.VMEM((K, DQ), jnp.float32),
            pltpu.VMEM((RB_CP, DQ), jnp.float32),
            pltpu.VMEM_SHARED((N_PAD, DQ), jnp.float32),
            pltpu.SemaphoreType.DMA,
            pltpu.SemaphoreType.DMA,
            pltpu.SemaphoreType.DMA,
            pltpu.SemaphoreType.DMA,
        ],
        compiler_params=pltpu.CompilerParams(use_tc_tiling_on_sc=False),
    )
    def k(h0_hbm, h1_hbm, h2_hbm, h3_hbm, src_hbm, dst_hbm, w_hbm,
          o0_hbm, o1_hbm, o2_hbm, o3_hbm,
          src_v, dst_v, w_v, gbuf0, gbuf1, sbuf0, sbuf1, buf_v, acc_sh,
          gsem0, gsem1, ssem0, ssem1):
        c = lax.axis_index("c")
        s = lax.axis_index("s")
        r0 = s * ROWS_PER_TILE

        pltpu.sync_copy(src_hbm.at[s], src_v)
        pltpu.sync_copy(dst_hbm.at[s], dst_v)
        pltpu.sync_copy(w_hbm.at[s], w_v)

        def zrow(i, carry):
            for j in range(DQ // L):
                buf_v[i, pl.ds(j * L, L)] = jnp.zeros((L,), jnp.float32)
            return carry
        lax.fori_loop(0, RB_CP, zrow, 0)

        def process(t, gbuf, sbuf):
            def mul(g, cc):
                wv16 = w_v[pl.ds(t * K + g * L, L)]
                for e in range(L):
                    wb = wv16[e]
                    i = g * L + e
                    for j in range(DQ // L):
                        sl = pl.ds(j * L, L)
                        sbuf[i, sl] = gbuf[i, sl] * wb
                return cc
            lax.fori_loop(0, K // L, mul, 0)

        def pipeline(h_q, o_q):
            def zhop(i, carry):
                pltpu.sync_copy(
                    buf_v, acc_sh.at[pl.ds(r0 + i * RB_CP, RB_CP)])
                return carry
            lax.fori_loop(0, N_CP, zhop, 0)
            plsc.subcore_barrier()

            pltpu.async_copy(h_q.at[src_v.at[0]], gbuf0, gsem0)
            pltpu.async_copy(h_q.at[src_v.at[1]], gbuf1, gsem1)

            def stage(t, gbuf, sbuf, gsem, ssem, u):
                pltpu.make_async_copy(
                    h_q.at[src_v.at[t]], gbuf, gsem).wait()

                @pl.when(u > 0)
                def _():
                    pltpu.make_async_copy(
                        sbuf, acc_sh.at[dst_v.at[t]], ssem).wait()
                process(t, gbuf, sbuf)

                @pl.when(t + 2 < NCHUNK)
                def _():
                    pltpu.async_copy(h_q.at[src_v.at[t + 2]], gbuf, gsem)
                pltpu.async_copy(sbuf, acc_sh.at[dst_v.at[t]], ssem, add=True)

            def pair(u, carry):
                t0 = 2 * u
                stage(t0, gbuf0, sbuf0, gsem0, ssem0, u)
                stage(t0 + 1, gbuf1, sbuf1, gsem1, ssem1, u)
                return carry
            lax.fori_loop(0, NPAIR, pair, 0)
            pltpu.make_async_copy(
                sbuf0, acc_sh.at[dst_v.at[NCHUNK - 2]], ssem0).wait()
            pltpu.make_async_copy(
                sbuf1, acc_sh.at[dst_v.at[NCHUNK - 1]], ssem1).wait()
            plsc.subcore_barrier()

            def cphop(i, carry):
                pltpu.sync_copy(
                    acc_sh.at[pl.ds(r0 + i * RB_CP, RB_CP)], buf_v)
                pltpu.sync_copy(
                    buf_v, o_q.at[pl.ds(r0 + i * RB_CP, RB_CP)])
                return carry
            lax.fori_loop(0, N_CP, cphop, 0)
            lax.fori_loop(0, RB_CP, zrow, 0)

        @pl.when(c == 0)
        def _():
            pipeline(h0_hbm, o0_hbm)
            pipeline(h1_hbm, o1_hbm)

        @pl.when(c == 1)
        def _():
            pipeline(h2_hbm, o2_hbm)
            pipeline(h3_hbm, o3_hbm)

    return k(h4[0], h4[1], h4[2], h4[3], src3, dst3, w2)


def kernel(x, edge_index, edge_weight, W, b):
    src = edge_index[0].astype(jnp.int32)
    dst = edge_index[1].astype(jnp.int32)
    pad = E_PAD - N_EDGES
    zi = jnp.zeros((pad,), jnp.int32)
    src3 = jnp.concatenate([src, zi]).reshape(NS, NCHUNK, K)
    dst3 = jnp.concatenate([dst, zi]).reshape(NS, NCHUNK, K)
    w2 = jnp.concatenate(
        [edge_weight, jnp.zeros((pad,), jnp.float32)]).reshape(NS, E_PER_TILE)
    wt4 = jnp.transpose(W.T.reshape(D_IN, 4, DQ), (1, 0, 2))
    h4 = _linear_quarters(x, wt4, b.reshape(4, 1, DQ))
    outs = _sc_aggregate(h4, src3, dst3, w2)
    return jnp.concatenate([o[:N_NODES] for o in outs], axis=1)
